# baseline (speedup 1.0000x reference)
import functools

import jax
import jax.numpy as jnp
from jax import lax
from jax.experimental import pallas as pl
from jax.experimental.pallas import tpu as pltpu
from jax.experimental.pallas import tpu_sc as plsc

_B, _M, _D, _K = 128, 2048, 256, 64
_EPS = 1e-8
_NEG = -3.0e38
_BR = 8


def _sim_body(q_ref, km_ref, sim_ref):
    q = q_ref[0]
    km = km_ref[0]
    qn = jnp.sqrt(jnp.sum(q * q))
    dims = (((1,), (1,)), ((), ()))
    dot = lax.dot_general(q, km, dims, precision=lax.Precision.HIGHEST,
                          preferred_element_type=jnp.float32)
    ones = jnp.ones((1, _D), dtype=jnp.float32)
    ss = lax.dot_general(ones, km * km, dims, precision=lax.Precision.HIGHEST,
                         preferred_element_type=jnp.float32)
    denom = jnp.maximum(jnp.sqrt(ss) * qn, _EPS)
    sim_ref[0] = dot / denom


def _sim(query, key_memory):
    return pl.pallas_call(
        _sim_body,
        grid=(_B,),
        in_specs=[
            pl.BlockSpec((1, 1, _D), lambda b: (b, 0, 0)),
            pl.BlockSpec((1, _M, _D), lambda b: (b, 0, 0)),
        ],
        out_specs=pl.BlockSpec((1, 1, _M), lambda b: (b, 0, 0)),
        out_shape=jax.ShapeDtypeStruct((_B, 1, _M), jnp.float32),
    )(query.reshape(_B, 1, _D), key_memory)


def _topk_body(sim_ref, iterf_ref, w_ref, fidx_ref):
    it = iterf_ref[:, :1]
    iota = lax.broadcasted_iota(jnp.int32, (_BR, _M), 1).astype(jnp.float32)
    s0 = jnp.where(iota <= it, sim_ref[...], 0.0)
    kio = lax.broadcasted_iota(jnp.int32, (_BR, _K), 1)

    def body(k, carry):
        s, vals, idxs = carry
        m = jnp.max(s, axis=1, keepdims=True)
        i = jnp.min(jnp.where(s == m, iota, 4096.0), axis=1, keepdims=True)
        s = jnp.where(iota == i, _NEG, s)
        sel = kio == k
        vals = jnp.where(sel, m, vals)
        idxs = jnp.where(sel, i, idxs)
        return s, vals, idxs

    zeros = jnp.zeros((_BR, _K), dtype=jnp.float32)
    _, vals, idxs = lax.fori_loop(0, _K, body, (s0, zeros, zeros))

    mx = jnp.max(vals, axis=1, keepdims=True)
    e = jnp.exp(vals - mx)
    w = e / jnp.sum(e, axis=1, keepdims=True)
    valid = (idxs <= it) & (it > 0.0)
    w_ref[...] = jnp.where(valid, w, 0.0)
    b0 = pl.program_id(0) * _BR
    brow = (b0 + lax.broadcasted_iota(jnp.int32, (_BR, _K), 0)).astype(jnp.float32)
    fidx_ref[...] = (brow * _M + idxs).astype(jnp.int32)


def _topk(sim, iterf):
    return pl.pallas_call(
        _topk_body,
        grid=(_B // _BR,),
        in_specs=[
            pl.BlockSpec((_BR, _M), lambda b: (b, 0)),
            pl.BlockSpec((_BR, 128), lambda b: (b, 0)),
        ],
        out_specs=[
            pl.BlockSpec((_BR, _K), lambda b: (b, 0)),
            pl.BlockSpec((_BR, _K), lambda b: (b, 0)),
        ],
        out_shape=[
            jax.ShapeDtypeStruct((_B, _K), jnp.float32),
            jax.ShapeDtypeStruct((_B, _K), jnp.int32),
        ],
    )(sim, iterf)


_NC, _NS = 2, 16
_NW = _NC * _NS
_RPW = (_B * _K) // _NW
_CH = 128
_NCH = _RPW // _CH


def _gather_body(table_ref, idx_ref, out_ref, idx_v, rows_v, sem):
    wid = lax.axis_index("s") * _NC + lax.axis_index("c")
    pltpu.sync_copy(idx_ref.at[wid], idx_v)
    for j in range(_NCH):
        pltpu.async_copy(table_ref.at[idx_v.at[j]], rows_v, sem).wait()
        pltpu.sync_copy(rows_v, out_ref.at[pl.ds(wid * _RPW + j * _CH, _CH)])


def _gather(table, idx3):
    f = functools.partial(
        pl.kernel,
        mesh=plsc.VectorSubcoreMesh(core_axis_name="c", subcore_axis_name="s"),
        out_type=jax.ShapeDtypeStruct((_B * _K, _D), jnp.float32),
        scratch_types=[
            pltpu.VMEM((_NCH, _CH), jnp.int32),
            pltpu.VMEM((_CH, _D), jnp.float32),
            pltpu.SemaphoreType.DMA,
        ],
    )(_gather_body)
    return f(table, idx3)


def _wsum_body(g_ref, w_ref, out_ref):
    g = g_ref[...]
    w = w_ref[...][..., None]
    out_ref[...] = jnp.sum(g * w, axis=1)


def _wsum(g, w):
    return pl.pallas_call(
        _wsum_body,
        grid=(_B // _BR,),
        in_specs=[
            pl.BlockSpec((_BR, _K, _D), lambda b: (b, 0, 0)),
            pl.BlockSpec((_BR, _K), lambda b: (b, 0)),
        ],
        out_specs=pl.BlockSpec((_BR, _D), lambda b: (b, 0)),
        out_shape=jax.ShapeDtypeStruct((_B, _D), jnp.float32),
    )(g, w)


def kernel(query, key_memory, value_memory, iteration):
    sim = _sim(query, key_memory).reshape(_B, _M)
    iterf = jnp.broadcast_to(iteration.astype(jnp.float32), (_B, 128))
    w, fidx = _topk(sim, iterf)
    table = value_memory.reshape(_B * _M, _D)
    idx3 = fidx.reshape(_NW, _NCH, _CH)
    g = _gather(table, idx3)
    return _wsum(g.reshape(_B, _K, _D), w)


# speedup vs baseline: 1.5910x; 1.5910x over previous
import functools

import jax
import jax.numpy as jnp
from jax import lax
from jax.experimental import pallas as pl
from jax.experimental.pallas import tpu as pltpu
from jax.experimental.pallas import tpu_sc as plsc

_B, _M, _D, _K = 128, 2048, 256, 64
_EPS = 1e-8
_NEG = -3.0e38
_BR = 8


def _sim_body(q_ref, km_ref, sim_ref):
    q = q_ref[0]
    km = km_ref[0]
    qn = jnp.sqrt(jnp.sum(q * q))
    dims = (((1,), (1,)), ((), ()))
    kh = km.astype(jnp.bfloat16)
    kl = (km - kh.astype(jnp.float32)).astype(jnp.bfloat16)
    qh = q.astype(jnp.bfloat16)
    ql = (q - qh.astype(jnp.float32)).astype(jnp.bfloat16)
    dot = (lax.dot_general(qh, kh, dims, preferred_element_type=jnp.float32)
           + lax.dot_general(ql, kh, dims, preferred_element_type=jnp.float32)
           + lax.dot_general(qh, kl, dims, preferred_element_type=jnp.float32))
    s = km * km
    sh = s.astype(jnp.bfloat16)
    sl = (s - sh.astype(jnp.float32)).astype(jnp.bfloat16)
    onesb = jnp.ones((1, _D), dtype=jnp.bfloat16)
    ss = (lax.dot_general(onesb, sh, dims, preferred_element_type=jnp.float32)
          + lax.dot_general(onesb, sl, dims, preferred_element_type=jnp.float32))
    denom = jnp.maximum(jnp.sqrt(ss) * qn, _EPS)
    sim_ref[0] = dot / denom


def _sim(query, key_memory):
    return pl.pallas_call(
        _sim_body,
        grid=(_B,),
        in_specs=[
            pl.BlockSpec((1, 1, _D), lambda b: (b, 0, 0)),
            pl.BlockSpec((1, _M, _D), lambda b: (b, 0, 0)),
        ],
        out_specs=pl.BlockSpec((1, 1, _M), lambda b: (b, 0, 0)),
        out_shape=jax.ShapeDtypeStruct((_B, 1, _M), jnp.float32),
    )(query.reshape(_B, 1, _D), key_memory)


def _topk_body(sim_ref, iterf_ref, w_ref, fidx_ref):
    it = iterf_ref[:, :1]
    iota = lax.broadcasted_iota(jnp.int32, (_BR, _M), 1).astype(jnp.float32)
    s0 = jnp.where(iota <= it, sim_ref[...], 0.0)
    kio = lax.broadcasted_iota(jnp.int32, (_BR, _K), 1)

    def body(k, carry):
        s, vals, idxs = carry
        m = jnp.max(s, axis=1, keepdims=True)
        i = jnp.min(jnp.where(s == m, iota, 4096.0), axis=1, keepdims=True)
        s = jnp.where(iota == i, _NEG, s)
        sel = kio == k
        vals = jnp.where(sel, m, vals)
        idxs = jnp.where(sel, i, idxs)
        return s, vals, idxs

    zeros = jnp.zeros((_BR, _K), dtype=jnp.float32)
    _, vals, idxs = lax.fori_loop(0, _K, body, (s0, zeros, zeros))

    mx = jnp.max(vals, axis=1, keepdims=True)
    e = jnp.exp(vals - mx)
    w = e / jnp.sum(e, axis=1, keepdims=True)
    valid = (idxs <= it) & (it > 0.0)
    w_ref[...] = jnp.where(valid, w, 0.0)
    b0 = pl.program_id(0) * _BR
    brow = (b0 + lax.broadcasted_iota(jnp.int32, (_BR, _K), 0)).astype(jnp.float32)
    fidx_ref[...] = (brow * _M + idxs).astype(jnp.int32)


def _topk(sim, iterf):
    return pl.pallas_call(
        _topk_body,
        grid=(_B // _BR,),
        in_specs=[
            pl.BlockSpec((_BR, _M), lambda b: (b, 0)),
            pl.BlockSpec((_BR, 128), lambda b: (b, 0)),
        ],
        out_specs=[
            pl.BlockSpec((_BR, _K), lambda b: (b, 0)),
            pl.BlockSpec((_BR, _K), lambda b: (b, 0)),
        ],
        out_shape=[
            jax.ShapeDtypeStruct((_B, _K), jnp.float32),
            jax.ShapeDtypeStruct((_B, _K), jnp.int32),
        ],
    )(sim, iterf)


_NC, _NS = 2, 16
_NW = _NC * _NS
_RPW = (_B * _K) // _NW
_CH = 128
_NCH = _RPW // _CH


def _gather_body(table_ref, idx_ref, out_ref, idx_v, rows_v, sem):
    wid = lax.axis_index("s") * _NC + lax.axis_index("c")
    pltpu.sync_copy(idx_ref.at[wid], idx_v)
    for j in range(_NCH):
        pltpu.async_copy(table_ref.at[idx_v.at[j]], rows_v, sem).wait()
        pltpu.sync_copy(rows_v, out_ref.at[pl.ds(wid * _RPW + j * _CH, _CH)])


def _gather(table, idx3):
    f = functools.partial(
        pl.kernel,
        mesh=plsc.VectorSubcoreMesh(core_axis_name="c", subcore_axis_name="s"),
        out_type=jax.ShapeDtypeStruct((_B * _K, _D), jnp.float32),
        scratch_types=[
            pltpu.VMEM((_NCH, _CH), jnp.int32),
            pltpu.VMEM((_CH, _D), jnp.float32),
            pltpu.SemaphoreType.DMA,
        ],
    )(_gather_body)
    return f(table, idx3)


def _wsum_body(g_ref, w_ref, out_ref):
    g = g_ref[...]
    w = w_ref[...][..., None]
    out_ref[...] = jnp.sum(g * w, axis=1)


def _wsum(g, w):
    return pl.pallas_call(
        _wsum_body,
        grid=(_B // _BR,),
        in_specs=[
            pl.BlockSpec((_BR, _K, _D), lambda b: (b, 0, 0)),
            pl.BlockSpec((_BR, _K), lambda b: (b, 0)),
        ],
        out_specs=pl.BlockSpec((_BR, _D), lambda b: (b, 0)),
        out_shape=jax.ShapeDtypeStruct((_B, _D), jnp.float32),
    )(g, w)


def kernel(query, key_memory, value_memory, iteration):
    sim = _sim(query, key_memory).reshape(_B, _M)
    iterf = jnp.broadcast_to(iteration.astype(jnp.float32), (_B, 128))
    w, fidx = _topk(sim, iterf)
    table = value_memory.reshape(_B * _M, _D)
    idx3 = fidx.reshape(_NW, _NCH, _CH)
    g = _gather(table, idx3)
    return _wsum(g.reshape(_B, _K, _D), w)


# speedup vs baseline: 2.4373x; 1.5319x over previous
import functools

import jax
import jax.numpy as jnp
from jax import lax
from jax.experimental import pallas as pl
from jax.experimental.pallas import tpu as pltpu
from jax.experimental.pallas import tpu_sc as plsc

_B, _M, _D, _K = 128, 2048, 256, 64
_EPS = 1e-8
_NEG = -3.0e38
_BR = 8
_BT = 128


def _sim_body(q_ref, km_ref, sim_ref):
    q = q_ref[0]
    km = km_ref[0]
    qn = jnp.sqrt(jnp.sum(q * q))
    dims = (((1,), (1,)), ((), ()))
    kh = km.astype(jnp.bfloat16)
    kl = (km - kh.astype(jnp.float32)).astype(jnp.bfloat16)
    qh = q.astype(jnp.bfloat16)
    ql = (q - qh.astype(jnp.float32)).astype(jnp.bfloat16)
    dot = (lax.dot_general(qh, kh, dims, preferred_element_type=jnp.float32)
           + lax.dot_general(ql, kh, dims, preferred_element_type=jnp.float32)
           + lax.dot_general(qh, kl, dims, preferred_element_type=jnp.float32))
    s = km * km
    sh = s.astype(jnp.bfloat16)
    sl = (s - sh.astype(jnp.float32)).astype(jnp.bfloat16)
    onesb = jnp.ones((1, _D), dtype=jnp.bfloat16)
    ss = (lax.dot_general(onesb, sh, dims, preferred_element_type=jnp.float32)
          + lax.dot_general(onesb, sl, dims, preferred_element_type=jnp.float32))
    denom = jnp.maximum(jnp.sqrt(ss) * qn, _EPS)
    sim_ref[0] = dot / denom


def _sim(query, key_memory):
    return pl.pallas_call(
        _sim_body,
        grid=(_B,),
        in_specs=[
            pl.BlockSpec((1, 1, _D), lambda b: (b, 0, 0)),
            pl.BlockSpec((1, _M, _D), lambda b: (b, 0, 0)),
        ],
        out_specs=pl.BlockSpec((1, 1, _M), lambda b: (b, 0, 0)),
        out_shape=jax.ShapeDtypeStruct((_B, 1, _M), jnp.float32),
    )(query.reshape(_B, 1, _D), key_memory)


def _topk_body(sim_ref, iterf_ref, w_ref, fidx_ref):
    it = iterf_ref[:, :1]
    iota = lax.broadcasted_iota(jnp.int32, (_BT, _M), 1).astype(jnp.float32)
    s0 = jnp.where(iota <= it, sim_ref[...], 0.0)
    kio = lax.broadcasted_iota(jnp.int32, (_BT, _K), 1)

    def body(k, carry):
        s, vals, idxs = carry
        m = jnp.max(s, axis=1, keepdims=True)
        i = jnp.min(jnp.where(s == m, iota, 4096.0), axis=1, keepdims=True)
        s = jnp.where(iota == i, _NEG, s)
        sel = kio == k
        vals = jnp.where(sel, m, vals)
        idxs = jnp.where(sel, i, idxs)
        return s, vals, idxs

    zeros = jnp.zeros((_BT, _K), dtype=jnp.float32)
    _, vals, idxs = lax.fori_loop(0, _K, body, (s0, zeros, zeros))

    mx = jnp.max(vals, axis=1, keepdims=True)
    e = jnp.exp(vals - mx)
    w = e / jnp.sum(e, axis=1, keepdims=True)
    valid = (idxs <= it) & (it > 0.0)
    w_ref[...] = jnp.where(valid, w, 0.0)
    b0 = pl.program_id(0) * _BT
    brow = (b0 + lax.broadcasted_iota(jnp.int32, (_BT, _K), 0)).astype(jnp.float32)
    fidx_ref[...] = (brow * _M + idxs).astype(jnp.int32)


def _topk(sim, iterf):
    return pl.pallas_call(
        _topk_body,
        grid=(_B // _BT,),
        in_specs=[
            pl.BlockSpec((_BT, _M), lambda b: (b, 0)),
            pl.BlockSpec((_BT, 128), lambda b: (b, 0)),
        ],
        out_specs=[
            pl.BlockSpec((_BT, _K), lambda b: (b, 0)),
            pl.BlockSpec((_BT, _K), lambda b: (b, 0)),
        ],
        out_shape=[
            jax.ShapeDtypeStruct((_B, _K), jnp.float32),
            jax.ShapeDtypeStruct((_B, _K), jnp.int32),
        ],
    )(sim, iterf)


_NC, _NS = 2, 16
_NW = _NC * _NS
_RPW = (_B * _K) // _NW
_CH = 128
_NCH = _RPW // _CH


def _gather_body(table_ref, idx_ref, out_ref, idx_v, rows_v, sem):
    wid = lax.axis_index("s") * _NC + lax.axis_index("c")
    pltpu.sync_copy(idx_ref.at[wid], idx_v)
    for j in range(_NCH):
        pltpu.async_copy(table_ref.at[idx_v.at[j]], rows_v, sem).wait()
        pltpu.sync_copy(rows_v, out_ref.at[pl.ds(wid * _RPW + j * _CH, _CH)])


def _gather(table, idx3):
    f = functools.partial(
        pl.kernel,
        mesh=plsc.VectorSubcoreMesh(core_axis_name="c", subcore_axis_name="s"),
        out_type=jax.ShapeDtypeStruct((_B * _K, _D), jnp.float32),
        scratch_types=[
            pltpu.VMEM((_NCH, _CH), jnp.int32),
            pltpu.VMEM((_CH, _D), jnp.float32),
            pltpu.SemaphoreType.DMA,
        ],
    )(_gather_body)
    return f(table, idx3)


def _wsum_body(g_ref, w_ref, out_ref):
    g = g_ref[...]
    w = w_ref[...][..., None]
    out_ref[...] = jnp.sum(g * w, axis=1)


def _wsum(g, w):
    return pl.pallas_call(
        _wsum_body,
        grid=(_B // _BR,),
        in_specs=[
            pl.BlockSpec((_BR, _K, _D), lambda b: (b, 0, 0)),
            pl.BlockSpec((_BR, _K), lambda b: (b, 0)),
        ],
        out_specs=pl.BlockSpec((_BR, _D), lambda b: (b, 0)),
        out_shape=jax.ShapeDtypeStruct((_B, _D), jnp.float32),
    )(g, w)


def kernel(query, key_memory, value_memory, iteration):
    sim = _sim(query, key_memory).reshape(_B, _M)
    iterf = jnp.broadcast_to(iteration.astype(jnp.float32), (_B, 128))
    w, fidx = _topk(sim, iterf)
    table = value_memory.reshape(_B * _M, _D)
    idx3 = fidx.reshape(_NW, _NCH, _CH)
    g = _gather(table, idx3)
    return _wsum(g.reshape(_B, _K, _D), w)


# speedup vs baseline: 2.4799x; 1.0175x over previous
import functools

import jax
import jax.numpy as jnp
from jax import lax
from jax.experimental import pallas as pl
from jax.experimental.pallas import tpu as pltpu
from jax.experimental.pallas import tpu_sc as plsc

_B, _M, _D, _K = 128, 2048, 256, 64
_EPS = 1e-8
_NEG = -3.0e38
_BR = 8
_BT = 128


def _sim_body(q_ref, km_ref, sim_ref):
    q = q_ref[0]
    km = km_ref[0]
    qn = jnp.sqrt(jnp.sum(q * q))
    dims = (((1,), (1,)), ((), ()))
    kh = km.astype(jnp.bfloat16)
    kl = (km - kh.astype(jnp.float32)).astype(jnp.bfloat16)
    qh = q.astype(jnp.bfloat16)
    ql = (q - qh.astype(jnp.float32)).astype(jnp.bfloat16)
    dot = (lax.dot_general(qh, kh, dims, preferred_element_type=jnp.float32)
           + lax.dot_general(ql, kh, dims, preferred_element_type=jnp.float32)
           + lax.dot_general(qh, kl, dims, preferred_element_type=jnp.float32))
    s = km * km
    sh = s.astype(jnp.bfloat16)
    sl = (s - sh.astype(jnp.float32)).astype(jnp.bfloat16)
    onesb = jnp.ones((1, _D), dtype=jnp.bfloat16)
    ss = (lax.dot_general(onesb, sh, dims, preferred_element_type=jnp.float32)
          + lax.dot_general(onesb, sl, dims, preferred_element_type=jnp.float32))
    denom = jnp.maximum(jnp.sqrt(ss) * qn, _EPS)
    sim_ref[0] = dot / denom


def _sim(query, key_memory):
    return pl.pallas_call(
        _sim_body,
        grid=(_B,),
        in_specs=[
            pl.BlockSpec((1, 1, _D), lambda b: (b, 0, 0)),
            pl.BlockSpec((1, _M, _D), lambda b: (b, 0, 0)),
        ],
        out_specs=pl.BlockSpec((1, 1, _M), lambda b: (b, 0, 0)),
        out_shape=jax.ShapeDtypeStruct((_B, 1, _M), jnp.float32),
    )(query.reshape(_B, 1, _D), key_memory)


def _topk_body(sim_ref, iterf_ref, w_ref, fidx_ref):
    it = iterf_ref[:, :1]
    iota = lax.broadcasted_iota(jnp.int32, (_BT, _M), 1).astype(jnp.float32)
    s0 = jnp.where(iota <= it, sim_ref[...], 0.0)
    kio = lax.broadcasted_iota(jnp.int32, (_BT, _K), 1)

    def body(k, carry):
        s, vals, idxs = carry
        m = jnp.max(s, axis=1, keepdims=True)
        i = jnp.min(jnp.where(s == m, iota, 4096.0), axis=1, keepdims=True)
        s = jnp.where(iota == i, _NEG, s)
        sel = kio == k
        vals = jnp.where(sel, m, vals)
        idxs = jnp.where(sel, i, idxs)
        return s, vals, idxs

    zeros = jnp.zeros((_BT, _K), dtype=jnp.float32)
    _, vals, idxs = lax.fori_loop(0, _K, body, (s0, zeros, zeros))

    mx = jnp.max(vals, axis=1, keepdims=True)
    e = jnp.exp(vals - mx)
    w = e / jnp.sum(e, axis=1, keepdims=True)
    valid = (idxs <= it) & (it > 0.0)
    w_ref[...] = jnp.where(valid, w, 0.0)
    b0 = pl.program_id(0) * _BT
    brow = (b0 + lax.broadcasted_iota(jnp.int32, (_BT, _K), 0)).astype(jnp.float32)
    fidx_ref[...] = (brow * _M + idxs).astype(jnp.int32)


def _topk(sim, iterf):
    return pl.pallas_call(
        _topk_body,
        grid=(_B // _BT,),
        in_specs=[
            pl.BlockSpec((_BT, _M), lambda b: (b, 0)),
            pl.BlockSpec((_BT, 128), lambda b: (b, 0)),
        ],
        out_specs=[
            pl.BlockSpec((_BT, _K), lambda b: (b, 0)),
            pl.BlockSpec((_BT, _K), lambda b: (b, 0)),
        ],
        out_shape=[
            jax.ShapeDtypeStruct((_B, _K), jnp.float32),
            jax.ShapeDtypeStruct((_B, _K), jnp.int32),
        ],
    )(sim, iterf)


_NC, _NS = 2, 16
_NW = _NC * _NS
_BPW = _B // _NW
_NDC = _D // 16


def _gather_wsum_body(table_ref, idx_ref, w_ref, out_ref, idx_v, wbuf, rows_v, obuf, sem):
    wid = lax.axis_index("s") * _NC + lax.axis_index("c")
    b0 = wid * _BPW
    pltpu.sync_copy(idx_ref.at[pl.ds(b0, _BPW)], idx_v)
    pltpu.sync_copy(w_ref.at[pl.ds(b0, _BPW)], wbuf)
    for bl in range(_BPW):
        pltpu.async_copy(table_ref.at[idx_v.at[bl]], rows_v, sem).wait()

        def kgroup(kg, acc):
            wv = wbuf[bl, pl.ds(kg * 16, 16)]
            for lane in range(16):
                k = kg * 16 + lane
                w16 = jnp.full((16,), wv[lane], dtype=jnp.float32)
                acc = tuple(acc[j] + w16 * rows_v[k, pl.ds(j * 16, 16)]
                            for j in range(_NDC))
            return acc

        acc0 = tuple(jnp.zeros((16,), jnp.float32) for _ in range(_NDC))
        acc = lax.fori_loop(0, _K // 16, kgroup, acc0)
        for j in range(_NDC):
            obuf[pl.ds(j * 16, 16)] = acc[j]
        pltpu.sync_copy(obuf, out_ref.at[b0 + bl])


def _gather_wsum(table, idx, w):
    f = functools.partial(
        pl.kernel,
        mesh=plsc.VectorSubcoreMesh(core_axis_name="c", subcore_axis_name="s"),
        out_type=jax.ShapeDtypeStruct((_B, _D), jnp.float32),
        scratch_types=[
            pltpu.VMEM((_BPW, _K), jnp.int32),
            pltpu.VMEM((_BPW, _K), jnp.float32),
            pltpu.VMEM((_K, _D), jnp.float32),
            pltpu.VMEM((_D,), jnp.float32),
            pltpu.SemaphoreType.DMA,
        ],
    )(_gather_wsum_body)
    return f(table, idx, w)


def kernel(query, key_memory, value_memory, iteration):
    sim = _sim(query, key_memory).reshape(_B, _M)
    iterf = jnp.broadcast_to(iteration.astype(jnp.float32), (_B, 128))
    w, fidx = _topk(sim, iterf)
    table = value_memory.reshape(_B * _M, _D)
    return _gather_wsum(table, fidx, w)


# speedup vs baseline: 2.6237x; 1.0580x over previous
import functools

import jax
import jax.numpy as jnp
from jax import lax
from jax.experimental import pallas as pl
from jax.experimental.pallas import tpu as pltpu
from jax.experimental.pallas import tpu_sc as plsc

_B, _M, _D, _K = 128, 2048, 256, 64
_EPS = 1e-8
_NEG = -3.0e38
_BR = 8
_BT = 128
_BS = 2


def _sim_body(q_ref, km_ref, sim_ref):
    dims = (((1,), (1,)), ((), ()))
    onesb = jnp.ones((1, _D), dtype=jnp.bfloat16)
    for bl in range(_BS):
        q = q_ref[bl]
        km = km_ref[bl]
        qn = jnp.sqrt(jnp.sum(q * q))
        kh = km.astype(jnp.bfloat16)
        kl = (km - kh.astype(jnp.float32)).astype(jnp.bfloat16)
        qh = q.astype(jnp.bfloat16)
        ql = (q - qh.astype(jnp.float32)).astype(jnp.bfloat16)
        dot = (lax.dot_general(qh, kh, dims, preferred_element_type=jnp.float32)
               + lax.dot_general(ql, kh, dims, preferred_element_type=jnp.float32)
               + lax.dot_general(qh, kl, dims, preferred_element_type=jnp.float32))
        s = km * km
        sh = s.astype(jnp.bfloat16)
        sl = (s - sh.astype(jnp.float32)).astype(jnp.bfloat16)
        ss = (lax.dot_general(onesb, sh, dims, preferred_element_type=jnp.float32)
              + lax.dot_general(onesb, sl, dims, preferred_element_type=jnp.float32))
        denom = jnp.maximum(jnp.sqrt(ss) * qn, _EPS)
        sim_ref[bl] = dot / denom


def _sim(query, key_memory):
    return pl.pallas_call(
        _sim_body,
        grid=(_B // _BS,),
        in_specs=[
            pl.BlockSpec((_BS, 1, _D), lambda b: (b, 0, 0)),
            pl.BlockSpec((_BS, _M, _D), lambda b: (b, 0, 0)),
        ],
        out_specs=pl.BlockSpec((_BS, 1, _M), lambda b: (b, 0, 0)),
        out_shape=jax.ShapeDtypeStruct((_B, 1, _M), jnp.float32),
    )(query.reshape(_B, 1, _D), key_memory)


def _topk_body(sim_ref, iterf_ref, w_ref, fidx_ref):
    it = iterf_ref[:, :1]
    iota = lax.broadcasted_iota(jnp.int32, (_BT, _M), 1).astype(jnp.float32)
    s0 = jnp.where(iota <= it, sim_ref[...], 0.0)
    kio = lax.broadcasted_iota(jnp.int32, (_BT, _K), 1)

    def body(k, carry):
        s, vals, idxs = carry
        m = jnp.max(s, axis=1, keepdims=True)
        i = jnp.min(jnp.where(s == m, iota, 4096.0), axis=1, keepdims=True)
        s = jnp.where(iota == i, _NEG, s)
        sel = kio == k
        vals = jnp.where(sel, m, vals)
        idxs = jnp.where(sel, i, idxs)
        return s, vals, idxs

    zeros = jnp.zeros((_BT, _K), dtype=jnp.float32)
    _, vals, idxs = lax.fori_loop(0, _K, body, (s0, zeros, zeros))

    mx = jnp.max(vals, axis=1, keepdims=True)
    e = jnp.exp(vals - mx)
    w = e / jnp.sum(e, axis=1, keepdims=True)
    valid = (idxs <= it) & (it > 0.0)
    w_ref[...] = jnp.where(valid, w, 0.0)
    b0 = pl.program_id(0) * _BT
    brow = (b0 + lax.broadcasted_iota(jnp.int32, (_BT, _K), 0)).astype(jnp.float32)
    fidx_ref[...] = (brow * _M + idxs).astype(jnp.int32)


def _topk(sim, iterf):
    return pl.pallas_call(
        _topk_body,
        grid=(_B // _BT,),
        in_specs=[
            pl.BlockSpec((_BT, _M), lambda b: (b, 0)),
            pl.BlockSpec((_BT, 128), lambda b: (b, 0)),
        ],
        out_specs=[
            pl.BlockSpec((_BT, _K), lambda b: (b, 0)),
            pl.BlockSpec((_BT, _K), lambda b: (b, 0)),
        ],
        out_shape=[
            jax.ShapeDtypeStruct((_B, _K), jnp.float32),
            jax.ShapeDtypeStruct((_B, _K), jnp.int32),
        ],
    )(sim, iterf)


_NC, _NS = 2, 16
_NW = _NC * _NS
_BPW = _B // _NW
_NDC = _D // 16


def _gather_wsum_body(table_ref, idx_ref, w_ref, out_ref, idx_v, wbuf, rows_v, obuf, sem):
    wid = lax.axis_index("s") * _NC + lax.axis_index("c")
    b0 = wid * _BPW
    pltpu.sync_copy(idx_ref.at[pl.ds(b0, _BPW)], idx_v)
    pltpu.sync_copy(w_ref.at[pl.ds(b0, _BPW)], wbuf)
    for bl in range(_BPW):
        pltpu.async_copy(table_ref.at[idx_v.at[bl]], rows_v, sem).wait()

        def kgroup(kg, acc):
            wv = wbuf[bl, pl.ds(kg * 16, 16)]
            for lane in range(16):
                k = kg * 16 + lane
                w16 = jnp.full((16,), wv[lane], dtype=jnp.float32)
                acc = tuple(acc[j] + w16 * rows_v[k, pl.ds(j * 16, 16)]
                            for j in range(_NDC))
            return acc

        acc0 = tuple(jnp.zeros((16,), jnp.float32) for _ in range(_NDC))
        acc = lax.fori_loop(0, _K // 16, kgroup, acc0)
        for j in range(_NDC):
            obuf[pl.ds(j * 16, 16)] = acc[j]
        pltpu.sync_copy(obuf, out_ref.at[b0 + bl])


def _gather_wsum(table, idx, w):
    f = functools.partial(
        pl.kernel,
        mesh=plsc.VectorSubcoreMesh(core_axis_name="c", subcore_axis_name="s"),
        out_type=jax.ShapeDtypeStruct((_B, _D), jnp.float32),
        scratch_types=[
            pltpu.VMEM((_BPW, _K), jnp.int32),
            pltpu.VMEM((_BPW, _K), jnp.float32),
            pltpu.VMEM((_K, _D), jnp.float32),
            pltpu.VMEM((_D,), jnp.float32),
            pltpu.SemaphoreType.DMA,
        ],
    )(_gather_wsum_body)
    return f(table, idx, w)


def kernel(query, key_memory, value_memory, iteration):
    sim = _sim(query, key_memory).reshape(_B, _M)
    iterf = jnp.broadcast_to(iteration.astype(jnp.float32), (_B, 128))
    w, fidx = _topk(sim, iterf)
    table = value_memory.reshape(_B * _M, _D)
    return _gather_wsum(table, fidx, w)


# speedup vs baseline: 2.6703x; 1.0178x over previous
import functools

import jax
import jax.numpy as jnp
from jax import lax
from jax.experimental import pallas as pl
from jax.experimental.pallas import tpu as pltpu
from jax.experimental.pallas import tpu_sc as plsc

_B, _M, _D, _K = 128, 2048, 256, 64
_EPS = 1e-8
_NEG = -3.0e38
_BR = 8
_BT = 128
_BS = 4


def _sim_body(q_ref, km_ref, sim_ref):
    dims = (((1,), (1,)), ((), ()))
    onesb = jnp.ones((1, _D), dtype=jnp.bfloat16)
    for bl in range(_BS):
        q = q_ref[bl]
        km = km_ref[bl]
        qn = jnp.sqrt(jnp.sum(q * q))
        kh = km.astype(jnp.bfloat16)
        kl = (km - kh.astype(jnp.float32)).astype(jnp.bfloat16)
        qh = q.astype(jnp.bfloat16)
        ql = (q - qh.astype(jnp.float32)).astype(jnp.bfloat16)
        dot = (lax.dot_general(qh, kh, dims, preferred_element_type=jnp.float32)
               + lax.dot_general(ql, kh, dims, preferred_element_type=jnp.float32)
               + lax.dot_general(qh, kl, dims, preferred_element_type=jnp.float32))
        s = km * km
        sh = s.astype(jnp.bfloat16)
        sl = (s - sh.astype(jnp.float32)).astype(jnp.bfloat16)
        ss = (lax.dot_general(onesb, sh, dims, preferred_element_type=jnp.float32)
              + lax.dot_general(onesb, sl, dims, preferred_element_type=jnp.float32))
        denom = jnp.maximum(jnp.sqrt(ss) * qn, _EPS)
        sim_ref[bl] = dot / denom


def _sim(query, key_memory):
    return pl.pallas_call(
        _sim_body,
        grid=(_B // _BS,),
        in_specs=[
            pl.BlockSpec((_BS, 1, _D), lambda b: (b, 0, 0)),
            pl.BlockSpec((_BS, _M, _D), lambda b: (b, 0, 0)),
        ],
        out_specs=pl.BlockSpec((_BS, 1, _M), lambda b: (b, 0, 0)),
        out_shape=jax.ShapeDtypeStruct((_B, 1, _M), jnp.float32),
    )(query.reshape(_B, 1, _D), key_memory)


def _topk_body(sim_ref, iterf_ref, w_ref, fidx_ref):
    it = iterf_ref[:, :1]
    iota = lax.broadcasted_iota(jnp.int32, (_BT, _M), 1).astype(jnp.float32)
    s0 = jnp.where(iota <= it, sim_ref[...], 0.0)
    kio = lax.broadcasted_iota(jnp.int32, (_BT, _K), 1)

    def body(k, carry):
        s, vals, idxs = carry
        m = jnp.max(s, axis=1, keepdims=True)
        i = jnp.min(jnp.where(s == m, iota, 4096.0), axis=1, keepdims=True)
        s = jnp.where(iota == i, _NEG, s)
        sel = kio == k
        vals = jnp.where(sel, m, vals)
        idxs = jnp.where(sel, i, idxs)
        return s, vals, idxs

    zeros = jnp.zeros((_BT, _K), dtype=jnp.float32)
    _, vals, idxs = lax.fori_loop(0, _K, body, (s0, zeros, zeros))

    mx = jnp.max(vals, axis=1, keepdims=True)
    e = jnp.exp(vals - mx)
    w = e / jnp.sum(e, axis=1, keepdims=True)
    valid = (idxs <= it) & (it > 0.0)
    w_ref[...] = jnp.where(valid, w, 0.0)
    b0 = pl.program_id(0) * _BT
    brow = (b0 + lax.broadcasted_iota(jnp.int32, (_BT, _K), 0)).astype(jnp.float32)
    fidx_ref[...] = (brow * _M + idxs).astype(jnp.int32)


def _topk(sim, iterf):
    return pl.pallas_call(
        _topk_body,
        grid=(_B // _BT,),
        in_specs=[
            pl.BlockSpec((_BT, _M), lambda b: (b, 0)),
            pl.BlockSpec((_BT, 128), lambda b: (b, 0)),
        ],
        out_specs=[
            pl.BlockSpec((_BT, _K), lambda b: (b, 0)),
            pl.BlockSpec((_BT, _K), lambda b: (b, 0)),
        ],
        out_shape=[
            jax.ShapeDtypeStruct((_B, _K), jnp.float32),
            jax.ShapeDtypeStruct((_B, _K), jnp.int32),
        ],
    )(sim, iterf)


_NC, _NS = 2, 16
_NW = _NC * _NS
_BPW = _B // _NW
_NDC = _D // 16


def _gather_wsum_body(table_ref, idx_ref, w_ref, out_ref, idx_v, wbuf, rows_v, obuf, sem):
    wid = lax.axis_index("s") * _NC + lax.axis_index("c")
    b0 = wid * _BPW
    pltpu.sync_copy(idx_ref.at[pl.ds(b0, _BPW)], idx_v)
    pltpu.sync_copy(w_ref.at[pl.ds(b0, _BPW)], wbuf)
    for bl in range(_BPW):
        pltpu.async_copy(table_ref.at[idx_v.at[bl]], rows_v, sem).wait()

        def kgroup(kg, acc):
            wv = wbuf[bl, pl.ds(kg * 16, 16)]
            for lane in range(16):
                k = kg * 16 + lane
                w16 = jnp.full((16,), wv[lane], dtype=jnp.float32)
                acc = tuple(acc[j] + w16 * rows_v[k, pl.ds(j * 16, 16)]
                            for j in range(_NDC))
            return acc

        acc0 = tuple(jnp.zeros((16,), jnp.float32) for _ in range(_NDC))
        acc = lax.fori_loop(0, _K // 16, kgroup, acc0)
        for j in range(_NDC):
            obuf[pl.ds(j * 16, 16)] = acc[j]
        pltpu.sync_copy(obuf, out_ref.at[b0 + bl])


def _gather_wsum(table, idx, w):
    f = functools.partial(
        pl.kernel,
        mesh=plsc.VectorSubcoreMesh(core_axis_name="c", subcore_axis_name="s"),
        out_type=jax.ShapeDtypeStruct((_B, _D), jnp.float32),
        scratch_types=[
            pltpu.VMEM((_BPW, _K), jnp.int32),
            pltpu.VMEM((_BPW, _K), jnp.float32),
            pltpu.VMEM((_K, _D), jnp.float32),
            pltpu.VMEM((_D,), jnp.float32),
            pltpu.SemaphoreType.DMA,
        ],
    )(_gather_wsum_body)
    return f(table, idx, w)


def kernel(query, key_memory, value_memory, iteration):
    sim = _sim(query, key_memory).reshape(_B, _M)
    iterf = jnp.broadcast_to(iteration.astype(jnp.float32), (_B, 128))
    w, fidx = _topk(sim, iterf)
    table = value_memory.reshape(_B * _M, _D)
    return _gather_wsum(table, fidx, w)
